# Initial kernel scaffold; baseline (speedup 1.0000x reference)
import jax
import jax.numpy as jnp
from jax.experimental import pallas as pl


def kernel(x_s, x_t, edge_index_s, edge_weight_s, edge_index_s1, edge_weight_s1, edge_index_s2, edge_weight_s2, idx_dic1, W0, b0, W1, b1, W2, b2, lin1_W, lin1_b, lin2_W, lin2_b, lin3_W, lin3_b):
    raise NotImplementedError("write your pallas kernel here")


# speedup vs baseline: 7.9272x; 7.9272x over previous
import functools

import jax
import jax.numpy as jnp
import numpy as np
from jax import lax
from jax.experimental import pallas as pl
from jax.experimental.pallas import tpu as pltpu
from jax.experimental.pallas import tpu_sc as plsc

ROI = 268
E1 = 4489
E2 = 2244
SLOPE = 0.33
BNS = float(1.0 / np.sqrt(1.0 + 1e-5))

B = 16
N0 = B * 8978
N1 = B * E1
N2 = B * E2
NP0 = 16 * 8984
NT = 2304


def _leaky(x):
    return jnp.where(x > 0, x, SLOPE * x)


def _pad_edges(ei, ew, mult):
    m = ei.shape[1]
    mp = ((m + mult - 1) // mult) * mult
    pad = mp - m
    src = jnp.pad(ei[0], (0, pad))
    dst = jnp.pad(ei[1], (0, pad))
    w = jnp.pad(ew, (0, pad))
    return src, dst, w, mp


_MESH = plsc.VectorSubcoreMesh(core_axis_name="c", subcore_axis_name="s")


@functools.lru_cache(maxsize=None)
def _make_prop_c16(n, mp):
    et = mp // 16
    nblk = et // 128
    rpt_a = ((n // 16 + 7) // 8) * 8
    rpt_l = n - 15 * rpt_a

    @functools.partial(
        pl.kernel, mesh=_MESH,
        compiler_params=pltpu.CompilerParams(use_tc_tiling_on_sc=False),
        out_type=jax.ShapeDtypeStruct((2 * n, 16), jnp.float32),
        scratch_types=[
            pltpu.VMEM((128,), jnp.int32),
            pltpu.VMEM((128,), jnp.int32),
            pltpu.VMEM((128,), jnp.float32),
            pltpu.VMEM((128, 16), jnp.float32),
            pltpu.VMEM((128, 16), jnp.float32),
            pltpu.VMEM_SHARED((n, 16), jnp.float32),
            pltpu.SemaphoreType.DMA,
        ],
    )
    def kfn(x_hbm, src_hbm, dst_hbm, w_hbm, out_hbm,
            srcb, dstb, wb, vals, zb, accum, sem):
        c = lax.axis_index("c")
        s = lax.axis_index("s")
        z16 = jnp.full((16,), 0.0, jnp.float32)
        for r in range(128):
            zb[r, :] = z16
        zbase = s * rpt_a

        def emit_zero(rows):
            nf, tl = rows // 128, rows % 128

            def zstep(i, _):
                pltpu.sync_copy(zb, accum.at[pl.ds(zbase + i * 128, 128), :])
                return 0
            lax.fori_loop(0, nf, zstep, 0)
            if tl:
                pltpu.sync_copy(zb.at[pl.ds(0, tl), :],
                                accum.at[pl.ds(zbase + nf * 128, tl), :])

        @pl.when(s < 15)
        def _():
            emit_zero(rpt_a)

        @pl.when(s == 15)
        def _():
            emit_zero(rpt_l)
        plsc.subcore_barrier()

        e_lo = s * et
        c_off = c * n

        def estep(i, _):
            base = e_lo + i * 128
            pltpu.sync_copy(src_hbm.at[pl.ds(base, 128)], srcb)
            pltpu.sync_copy(dst_hbm.at[pl.ds(base, 128)], dstb)
            pltpu.sync_copy(w_hbm.at[pl.ds(base, 128)], wb)
            for g in range(8):
                sl = pl.ds(g * 16, 16)
                srcb[sl] = srcb[sl] + c_off
            pltpu.async_copy(x_hbm.at[srcb], vals, sem).wait()
            for g in range(8):
                wv = wb[pl.ds(g * 16, 16)]
                for i in range(16):
                    e = g * 16 + i
                    vals[e, :] = vals[e, :] * wv[i]
            pltpu.sync_copy(vals, accum.at[dstb], add=True)
            return 0
        lax.fori_loop(0, nblk, estep, 0)
        plsc.subcore_barrier()

        obase = c_off + zbase

        def emit_out(rows):
            nf, tl = rows // 128, rows % 128

            def ostep(i, _):
                pltpu.sync_copy(accum.at[pl.ds(zbase + i * 128, 128), :], zb)
                pltpu.sync_copy(zb, out_hbm.at[pl.ds(obase + i * 128, 128), :])
                return 0
            lax.fori_loop(0, nf, ostep, 0)
            if tl:
                pltpu.sync_copy(accum.at[pl.ds(zbase + nf * 128, tl), :],
                                zb.at[pl.ds(0, tl), :])
                pltpu.sync_copy(zb.at[pl.ds(0, tl), :],
                                out_hbm.at[pl.ds(obase + nf * 128, tl), :])

        @pl.when(s < 15)
        def _():
            emit_out(rpt_a)

        @pl.when(s == 15)
        def _():
            emit_out(rpt_l)

    return kfn


@functools.lru_cache(maxsize=None)
def _make_prop_c1(np_len, mp):
    ew_per = mp // 32
    nblk = ew_per // 128
    wpt = np_len // 16
    nz_full, nz_tail = wpt // 2048, wpt % 2048

    @functools.partial(
        pl.kernel, mesh=_MESH,
        compiler_params=pltpu.CompilerParams(use_tc_tiling_on_sc=False),
        out_type=jax.ShapeDtypeStruct((2 * np_len,), jnp.float32),
        scratch_types=[
            pltpu.VMEM((128,), jnp.int32),
            pltpu.VMEM((128,), jnp.int32),
            pltpu.VMEM((128,), jnp.float32),
            pltpu.VMEM((128,), jnp.float32),
            pltpu.VMEM((2048,), jnp.float32),
            pltpu.VMEM_SHARED((np_len,), jnp.float32),
            pltpu.SemaphoreType.DMA,
        ],
    )
    def kfn(x_hbm, src_hbm, dst_hbm, w_hbm, out_hbm,
            srcb, dstb, wb, vals, zb, accum, sem):
        c = lax.axis_index("c")
        s = lax.axis_index("s")
        z16 = jnp.full((16,), 0.0, jnp.float32)
        for r in range(128):
            zb[pl.ds(r * 16, 16)] = z16
        zbase = s * wpt

        def zstep(i, _):
            pltpu.sync_copy(zb, accum.at[pl.ds(zbase + i * 2048, 2048)])
            return 0
        lax.fori_loop(0, nz_full, zstep, 0)
        if nz_tail:
            pltpu.sync_copy(zb.at[pl.ds(0, nz_tail)],
                            accum.at[pl.ds(zbase + nz_full * 2048, nz_tail)])
        plsc.subcore_barrier()

        e_lo = (c * 16 + s) * ew_per

        def estep(i, _):
            base = e_lo + i * 128
            pltpu.sync_copy(src_hbm.at[pl.ds(base, 128)], srcb)
            pltpu.sync_copy(dst_hbm.at[pl.ds(base, 128)], dstb)
            pltpu.sync_copy(w_hbm.at[pl.ds(base, 128)], wb)
            pltpu.async_copy(x_hbm.at[srcb], vals, sem).wait()
            for g in range(8):
                sl = pl.ds(g * 16, 16)
                vals[sl] = vals[sl] * wb[sl]
            pltpu.sync_copy(vals, accum.at[dstb], add=True)
            return 0
        lax.fori_loop(0, nblk, estep, 0)
        plsc.subcore_barrier()

        ob = c * np_len + zbase

        def ostep(i, _):
            pltpu.sync_copy(accum.at[pl.ds(zbase + i * 2048, 2048)], zb)
            pltpu.sync_copy(zb, out_hbm.at[pl.ds(ob + i * 2048, 2048)])
            return 0
        lax.fori_loop(0, nz_full, ostep, 0)
        if nz_tail:
            pltpu.sync_copy(accum.at[pl.ds(zbase + nz_full * 2048, nz_tail)],
                            zb.at[pl.ds(0, nz_tail)])
            pltpu.sync_copy(zb.at[pl.ds(0, nz_tail)],
                            out_hbm.at[pl.ds(ob + nz_full * 2048, nz_tail)])

    return kfn


@functools.lru_cache(maxsize=None)
def _make_permpool(n_in):
    nfull = 17
    tail = E2 - nfull * 128

    @functools.partial(
        pl.kernel, mesh=_MESH,
        compiler_params=pltpu.CompilerParams(use_tc_tiling_on_sc=False),
        out_type=[
            jax.ShapeDtypeStruct((2 * 16 * NT, 16), jnp.float32),
            jax.ShapeDtypeStruct((512,), jnp.float32),
        ],
        scratch_types=[
            pltpu.VMEM((128,), jnp.int32),
            pltpu.VMEM((128,), jnp.int32),
            pltpu.VMEM((128, 16), jnp.float32),
            pltpu.VMEM((128, 16), jnp.float32),
            pltpu.VMEM((128, 16), jnp.float32),
            pltpu.VMEM((16,), jnp.float32),
            pltpu.SemaphoreType.DMA,
        ],
    )
    def kfn(x_hbm, srca_hbm, srcb_hbm, out_hbm, g1_hbm,
            sa, sb, ra, rb, vals, g1b, sem):
        c = lax.axis_index("c")
        s = lax.axis_index("s")
        c_off = c * n_in

        def do_chunk(i, nacc, gacc):
            pltpu.sync_copy(srca_hbm.at[pl.ds(s * NT + i * 128, 128)], sa)
            pltpu.sync_copy(srcb_hbm.at[pl.ds(s * NT + i * 128, 128)], sb)
            for g in range(8):
                sl = pl.ds(g * 16, 16)
                sa[sl] = sa[sl] + c_off
                sb[sl] = sb[sl] + c_off
            pltpu.async_copy(x_hbm.at[sa], ra, sem).wait()
            pltpu.async_copy(x_hbm.at[sb], rb, sem).wait()
            for e in range(128):
                row = (ra[e, :] + rb[e, :]) * 0.5
                vals[e, :] = row
                if e < nacc:
                    gacc = gacc + row
            pltpu.sync_copy(
                vals, out_hbm.at[pl.ds((c * 16 + s) * NT + i * 128, 128), :])
            return gacc

        def step(i, gacc):
            return do_chunk(i, 128, gacc)
        gacc = lax.fori_loop(0, nfull, step, jnp.full((16,), 0.0, jnp.float32))
        gacc = do_chunk(nfull, tail, gacc)
        g1b[...] = gacc * (1.0 / E2)
        pltpu.sync_copy(g1b, g1_hbm.at[pl.ds((c * 16 + s) * 16, 16)])

    return kfn


def _comb0_body(a, bb, inv, tc_ref, tp_ref, p_ref, o_ref):
    p = p_ref[...]
    p0 = p[:1123]
    p1 = p[1123:]
    o_ref[...] = (a * tc_ref[...] - p0 - p1 - bb * tp_ref[...]) * inv


def _comb0(tc, tp, p, a, bb, inv):
    body = functools.partial(_comb0_body, a, bb, inv)
    out = pl.pallas_call(
        body, out_shape=jax.ShapeDtypeStruct((1123, 128), jnp.float32),
    )(tc.reshape(1123, 128), tp.reshape(1123, 128), p.reshape(2246, 128))
    return out.reshape(NP0)


def _comb_body(a, bb, inv, tc_ref, tp_ref, p_ref, o_ref):
    o_ref[...] = (a * tc_ref[...] - p_ref[...] - bb * tp_ref[...]) * inv


def _comb16(tc, tp, p, a, bb, inv, rows):
    body = functools.partial(_comb_body, a, bb, inv)
    out = pl.pallas_call(
        body, out_shape=jax.ShapeDtypeStruct((rows, 128), jnp.float32),
    )(tc.reshape(rows, 128), tp.reshape(rows, 128), p.reshape(rows, 128))
    return out.reshape(rows * 8, 16)


def _conv0_body(t0_ref, t1_ref, t2_ref, t3_ref, eev_ref, eod_ref, b_ref,
                o_ref, g_ref):
    i = pl.program_id(1)
    t4 = jnp.concatenate(
        [t0_ref[...], t1_ref[...], t2_ref[...], t3_ref[...]], axis=1)
    bias = b_ref[0][0:1, :]
    ve = jnp.dot(t4, eev_ref[0], preferred_element_type=jnp.float32) + bias
    vo = jnp.dot(t4, eod_ref[0], preferred_element_type=jnp.float32) + bias
    o = 0.5 * (_leaky(BNS * ve) + _leaky(BNS * vo))
    o_ref[0] = o
    rows = o.shape[0]
    r_iota = lax.broadcasted_iota(jnp.int32, (rows, 128), 0)
    l_iota = lax.broadcasted_iota(jnp.int32, (rows, 128), 1)
    p = (i * rows + r_iota) * 8 + l_iota // 16
    bid = p // E1
    acc = jnp.stack(
        [jnp.where(bid == b, o, 0.0).sum(axis=0) for b in range(16)], axis=0)

    @pl.when(i == 0)
    def _():
        g_ref[0] = acc

    @pl.when(i > 0)
    def _():
        g_ref[0] = g_ref[0] + acc


def _conv0_matmul(t0, t1, t2, t3, eev, eod, b128):
    bm = 1024
    nb = (8978 + bm - 1) // bm
    grid = (2, nb)
    tspec = pl.BlockSpec((bm, 16), lambda h, i: (i, 0))
    espec = pl.BlockSpec((1, 64, 128), lambda h, i: (h, 0, 0))
    return pl.pallas_call(
        _conv0_body,
        grid=grid,
        in_specs=[tspec, tspec, tspec, tspec, espec, espec,
                  pl.BlockSpec((1, 8, 128), lambda h, i: (h, 0, 0))],
        out_specs=[pl.BlockSpec((1, bm, 128), lambda h, i: (h, i, 0)),
                   pl.BlockSpec((1, 16, 128), lambda h, i: (h, 0, 0))],
        out_shape=[jax.ShapeDtypeStruct((2, 8978, 128), jnp.float32),
                   jax.ShapeDtypeStruct((2, 16, 128), jnp.float32)],
    )(t0.reshape(8984, 16), t1.reshape(8984, 16), t2.reshape(8984, 16),
      t3.reshape(8984, 16), eev, eod, b128)


def _conv1_body(th_refs, w_ref, b_ref, o_ref):
    t128 = jnp.concatenate([r[0] for r in th_refs], axis=1)
    z = jnp.dot(t128, w_ref[0], preferred_element_type=jnp.float32) + b_ref[0][0:1, :]
    o_ref[0] = _leaky(BNS * z)


def _conv1_matmul(ts, w1rh, b1h):
    bm = 1024
    nb = (N1 + bm - 1) // bm
    grid = (2, nb)
    specs = []
    ops = []
    for t in ts:
        tv = t.reshape(2, N1, 16)
        for h in range(2):
            ops.append(tv)
            specs.append(pl.BlockSpec(
                (1, bm, 16), functools.partial(
                    lambda hh, h_, i_: (hh, i_, 0), h)))
    body = lambda *refs: _conv1_body(refs[:8], refs[8], refs[9], refs[10])
    out = pl.pallas_call(
        body,
        grid=grid,
        in_specs=specs + [pl.BlockSpec((1, 128, 16), lambda h, i: (h, 0, 0)),
                          pl.BlockSpec((1, 8, 16), lambda h, i: (h, 0, 0))],
        out_specs=pl.BlockSpec((1, bm, 16), lambda h, i: (h, i, 0)),
        out_shape=jax.ShapeDtypeStruct((2, N1, 16), jnp.float32),
    )(*ops, w1rh, b1h)
    return out.reshape(2 * N1, 16)


def _conv2_body(refs):
    t128 = jnp.concatenate([r[0] for r in refs[:8]], axis=1)
    z = jnp.dot(t128, refs[8][...], preferred_element_type=jnp.float32)
    refs[10][...] = _leaky(BNS * (z + refs[9][...]))


def _conv2_matmul(ts, w128, b2):
    bm = 1024
    nb = (N2 + bm - 1) // bm
    specs = []
    ops = []
    for t in ts:
        tv = t.reshape(2, N2, 16)
        for h in range(2):
            ops.append(tv)
            specs.append(pl.BlockSpec(
                (1, bm, 16), functools.partial(
                    lambda hh, i_: (hh, i_, 0), h)))
    body = lambda *refs: _conv2_body(refs)
    return pl.pallas_call(
        body,
        grid=(nb,),
        in_specs=specs + [pl.BlockSpec((128, 1), lambda i: (0, 0)),
                          pl.BlockSpec((1, 1), lambda i: (0, 0))],
        out_specs=pl.BlockSpec((bm, 1), lambda i: (i, 0)),
        out_shape=jax.ShapeDtypeStruct((N2, 1), jnp.float32),
    )(*ops, w128, b2.reshape(1, 1))


def _mlp_body(x_ref, w1_ref, b1_ref, w2_ref, b2_ref, w3_ref, b3_ref, o_ref):
    x = x_ref[...]
    z = jnp.dot(x, w1_ref[...], preferred_element_type=jnp.float32) + b1_ref[...]
    h = jnp.maximum(z * BNS, 0.0)
    z = jnp.dot(h, w2_ref[...], preferred_element_type=jnp.float32) + b2_ref[...]
    h = jnp.maximum(z * BNS, 0.0)
    o_ref[...] = jnp.dot(h, w3_ref[...], preferred_element_type=jnp.float32) + b3_ref[...]


def _mlp(x, w1, b1, w2, b2, w3, b3):
    return pl.pallas_call(
        _mlp_body,
        out_shape=jax.ShapeDtypeStruct((x.shape[0], 1), jnp.float32),
    )(x, w1, b1[None, :], w2, b2[None, :], w3, b3[None, :])


_MASK_EV = np.zeros((16, 8), np.float32)
_MASK_OD = np.zeros((16, 8), np.float32)
for _j in range(8):
    _MASK_EV[2 * _j, _j] = 1.0
    _MASK_OD[2 * _j + 1, _j] = 1.0


def _conv0_consts(W0, b0):
    w0h = W0[:, 0, :].reshape(4, 2, 16).transpose(1, 0, 2)
    eev = (_MASK_EV[None, None, :, :, None]
           * w0h[:, :, None, None, :]).reshape(2, 64, 128)
    eod = (_MASK_OD[None, None, :, :, None]
           * w0h[:, :, None, None, :]).reshape(2, 64, 128)
    b128 = jnp.tile(b0.reshape(2, 1, 16), (1, 8, 1)).reshape(2, 1, 128)
    b128 = jnp.tile(b128, (1, 8, 1))
    return eev, eod, b128


def kernel(x_s, x_t, edge_index_s, edge_weight_s, edge_index_s1, edge_weight_s1,
           edge_index_s2, edge_weight_s2, idx_dic1,
           W0, b0, W1, b1, W2, b2,
           lin1_W, lin1_b, lin2_W, lin2_b, lin3_W, lin3_b):
    x0 = jnp.pad(x_s[:, 0], (0, NP0 - N0))
    src0, dst0, w0, mp0 = _pad_edges(edge_index_s, edge_weight_s, 4096)
    prop0 = _make_prop_c1(NP0, mp0)

    pa = prop0(x0, src0, dst0, w0)
    t1 = _comb0(x0, x0, pa, 1.0, 0.0, 1.0)
    pb = prop0(t1, src0, dst0, w0)
    t2 = _comb0(t1, x0, pb, 3.0, 1.0, 0.5)
    pc = prop0(t2, src0, dst0, w0)
    t3 = _comb0(t2, t1, pc, 5.0, 2.0, 1.0 / 3.0)

    eev, eod, b128 = _conv0_consts(W0, b0)
    h0p, g0p = _conv0_matmul(x0, t1, t2, t3, eev, eod, b128)
    g0 = (g0p.reshape(2, 16, 8, 16).sum(axis=2) / E1)
    g0 = jnp.concatenate([g0[0], g0[1]], axis=-1)

    x1 = h0p.reshape(2 * N1, 16)
    src1, dst1, w1e, mp1 = _pad_edges(edge_index_s1, edge_weight_s1, 2048)
    prop1 = _make_prop_c16(N1, mp1)
    rows1 = (2 * N1 * 16) // 128

    pa = prop1(x1, src1, dst1, w1e)
    t1 = _comb16(x1.reshape(rows1, 128), x1.reshape(rows1, 128), pa,
                 1.0, 0.0, 1.0, rows1)
    pb = prop1(t1, src1, dst1, w1e)
    t2 = _comb16(t1.reshape(rows1, 128), x1.reshape(rows1, 128), pb,
                 3.0, 1.0, 0.5, rows1)
    pc = prop1(t2, src1, dst1, w1e)
    t3 = _comb16(t2.reshape(rows1, 128), t1.reshape(rows1, 128), pc,
                 5.0, 2.0, 1.0 / 3.0, rows1)

    w1rh = jnp.swapaxes(W1.reshape(128, 2, 16), 0, 1)
    b1h = jnp.tile(b1.reshape(2, 1, 16), (1, 8, 1))
    h1 = _conv1_matmul([x1, t1, t2, t3], w1rh, b1h)

    ev = idx_dic1[0:2 * E2:2]
    od = idx_dic1[1:2 * E2:2]
    boff = jnp.arange(16, dtype=jnp.int32)[:, None] * E1
    srca = jnp.pad((boff + ev[None, :]).astype(jnp.int32),
                   ((0, 0), (0, NT - E2)))
    srcb = jnp.pad((boff + od[None, :]).astype(jnp.int32),
                   ((0, 0), (0, NT - E2)))
    h1p_pad, g1h = _make_permpool(N1)(h1, srca.reshape(-1), srcb.reshape(-1))
    g1h = g1h.reshape(2, 16, 16)
    g1 = jnp.concatenate([g1h[0], g1h[1]], axis=-1)
    x2 = h1p_pad.reshape(2, 16, NT, 16)[:, :, :E2, :].reshape(2 * N2, 16)

    src2, dst2, w2e, mp2 = _pad_edges(edge_index_s2, edge_weight_s2, 2048)
    prop2 = _make_prop_c16(N2, mp2)
    rows2 = (2 * N2 * 16) // 128

    pa = prop2(x2, src2, dst2, w2e)
    t1 = _comb16(x2.reshape(rows2, 128), x2.reshape(rows2, 128), pa,
                 1.0, 0.0, 1.0, rows2)
    pb = prop2(t1, src2, dst2, w2e)
    t2 = _comb16(t1.reshape(rows2, 128), x2.reshape(rows2, 128), pb,
                 3.0, 1.0, 0.5, rows2)
    pc = prop2(t2, src2, dst2, w2e)
    t3 = _comb16(t2.reshape(rows2, 128), t1.reshape(rows2, 128), pc,
                 5.0, 2.0, 1.0 / 3.0, rows2)

    w128 = W2.reshape(128, 1)
    h2 = _conv2_matmul([x2, t1, t2, t3], w128, b2)

    x = jnp.concatenate([h2.reshape(B, E2), g0, g1], axis=-1)
    return _mlp(x, lin1_W, lin1_b, lin2_W, lin2_b, lin3_W, lin3_b)


# speedup vs baseline: 19.2099x; 2.4233x over previous
import functools

import jax
import jax.numpy as jnp
import numpy as np
from jax import lax
from jax.experimental import pallas as pl
from jax.experimental.pallas import tpu as pltpu
from jax.experimental.pallas import tpu_sc as plsc

ROI = 268
E1 = 4489
E2 = 2244
SLOPE = 0.33
BNS = float(1.0 / np.sqrt(1.0 + 1e-5))

B = 16
N0 = B * 8978
N1 = B * E1
N2 = B * E2
NP0 = 16 * 8984
NT = 2304


def _leaky(x):
    return jnp.where(x > 0, x, SLOPE * x)


def _pad_edges(ei, ew, mult):
    m = ei.shape[1]
    mp = ((m + mult - 1) // mult) * mult
    pad = mp - m
    src = jnp.pad(ei[0], (0, pad))
    dst = jnp.pad(ei[1], (0, pad))
    w = jnp.pad(ew, (0, pad))
    return src, dst, w, mp


_MESH = plsc.VectorSubcoreMesh(core_axis_name="c", subcore_axis_name="s")


@functools.lru_cache(maxsize=None)
def _make_prop_c16(n, mp):
    et = mp // 16
    nch = et // 1024
    rpt_a = ((n // 16 + 7) // 8) * 8
    rpt_l = n - 15 * rpt_a

    @functools.partial(
        pl.kernel, mesh=_MESH,
        compiler_params=pltpu.CompilerParams(use_tc_tiling_on_sc=False),
        out_type=jax.ShapeDtypeStruct((2 * n, 16), jnp.float32),
        scratch_types=[
            pltpu.VMEM((1024,), jnp.int32),
            pltpu.VMEM((1024,), jnp.int32),
            pltpu.VMEM((1024,), jnp.float32),
            pltpu.VMEM((1024, 16), jnp.float32),
            pltpu.VMEM((128, 16), jnp.float32),
            pltpu.VMEM_SHARED((n, 16), jnp.float32),
            pltpu.SemaphoreType.DMA,
        ],
    )
    def kfn(x_hbm, src_hbm, dst_hbm, w_hbm, out_hbm,
            srcb, dstb, wb, vals, zb, accum, sem):
        c = lax.axis_index("c")
        s = lax.axis_index("s")
        z16 = jnp.full((16,), 0.0, jnp.float32)
        for r in range(128):
            zb[r, :] = z16
        zbase = s * rpt_a

        def emit_zero(rows):
            nf, tl = rows // 128, rows % 128

            def zstep(i, _):
                pltpu.sync_copy(zb, accum.at[pl.ds(zbase + i * 128, 128), :])
                return 0
            lax.fori_loop(0, nf, zstep, 0)
            if tl:
                pltpu.sync_copy(zb.at[pl.ds(0, tl), :],
                                accum.at[pl.ds(zbase + nf * 128, tl), :])

        @pl.when(s < 15)
        def _():
            emit_zero(rpt_a)

        @pl.when(s == 15)
        def _():
            emit_zero(rpt_l)
        plsc.subcore_barrier()

        e_lo = s * et
        c_off = c * n

        def estep(i, _):
            base = e_lo + i * 1024
            pltpu.sync_copy(src_hbm.at[pl.ds(base, 1024)], srcb)
            pltpu.sync_copy(dst_hbm.at[pl.ds(base, 1024)], dstb)
            pltpu.sync_copy(w_hbm.at[pl.ds(base, 1024)], wb)
            for g in range(64):
                sl = pl.ds(g * 16, 16)
                srcb[sl] = srcb[sl] + c_off
            pltpu.async_copy(x_hbm.at[srcb], vals, sem).wait()

            def scale_j(j, _):
                jb = j * 128
                for g in range(8):
                    wv = wb[pl.ds(jb + g * 16, 16)]
                    for i2 in range(16):
                        vals[jb + g * 16 + i2, :] = (
                            vals[jb + g * 16 + i2, :] * wv[i2])
                return 0
            lax.fori_loop(0, 8, scale_j, 0)
            pltpu.sync_copy(vals, accum.at[dstb], add=True)
            return 0
        lax.fori_loop(0, nch, estep, 0)
        plsc.subcore_barrier()

        obase = c_off + zbase

        def emit_out(rows):
            nf, tl = rows // 128, rows % 128

            def ostep(i, _):
                pltpu.sync_copy(accum.at[pl.ds(zbase + i * 128, 128), :], zb)
                pltpu.sync_copy(zb, out_hbm.at[pl.ds(obase + i * 128, 128), :])
                return 0
            lax.fori_loop(0, nf, ostep, 0)
            if tl:
                pltpu.sync_copy(accum.at[pl.ds(zbase + nf * 128, tl), :],
                                zb.at[pl.ds(0, tl), :])
                pltpu.sync_copy(zb.at[pl.ds(0, tl), :],
                                out_hbm.at[pl.ds(obase + nf * 128, tl), :])

        @pl.when(s < 15)
        def _():
            emit_out(rpt_a)

        @pl.when(s == 15)
        def _():
            emit_out(rpt_l)

    return kfn


@functools.lru_cache(maxsize=None)
def _make_prop_c1(np_len, mp):
    ew_per = mp // 32
    nch = ew_per // 1024
    wpt = np_len // 16
    nz_full, nz_tail = wpt // 2048, wpt % 2048

    @functools.partial(
        pl.kernel, mesh=_MESH,
        compiler_params=pltpu.CompilerParams(use_tc_tiling_on_sc=False),
        out_type=jax.ShapeDtypeStruct((2 * np_len,), jnp.float32),
        scratch_types=[
            pltpu.VMEM((1024,), jnp.int32),
            pltpu.VMEM((1024,), jnp.int32),
            pltpu.VMEM((1024,), jnp.float32),
            pltpu.VMEM((1024,), jnp.float32),
            pltpu.VMEM((2048,), jnp.float32),
            pltpu.VMEM_SHARED((np_len,), jnp.float32),
            pltpu.SemaphoreType.DMA,
        ],
    )
    def kfn(x_hbm, src_hbm, dst_hbm, w_hbm, out_hbm,
            srcb, dstb, wb, vals, zb, accum, sem):
        c = lax.axis_index("c")
        s = lax.axis_index("s")
        z16 = jnp.full((16,), 0.0, jnp.float32)
        for r in range(128):
            zb[pl.ds(r * 16, 16)] = z16
        zbase = s * wpt

        def zstep(i, _):
            pltpu.sync_copy(zb, accum.at[pl.ds(zbase + i * 2048, 2048)])
            return 0
        lax.fori_loop(0, nz_full, zstep, 0)
        if nz_tail:
            pltpu.sync_copy(zb.at[pl.ds(0, nz_tail)],
                            accum.at[pl.ds(zbase + nz_full * 2048, nz_tail)])
        plsc.subcore_barrier()

        e_lo = (c * 16 + s) * ew_per

        def estep(i, _):
            base = e_lo + i * 1024
            pltpu.sync_copy(src_hbm.at[pl.ds(base, 1024)], srcb)
            pltpu.sync_copy(dst_hbm.at[pl.ds(base, 1024)], dstb)
            pltpu.sync_copy(w_hbm.at[pl.ds(base, 1024)], wb)
            pltpu.async_copy(x_hbm.at[srcb], vals, sem).wait()
            for g in range(64):
                sl = pl.ds(g * 16, 16)
                vals[sl] = vals[sl] * wb[sl]
            pltpu.sync_copy(vals, accum.at[dstb], add=True)
            return 0
        lax.fori_loop(0, nch, estep, 0)
        plsc.subcore_barrier()

        ob = c * np_len + zbase

        def ostep(i, _):
            pltpu.sync_copy(accum.at[pl.ds(zbase + i * 2048, 2048)], zb)
            pltpu.sync_copy(zb, out_hbm.at[pl.ds(ob + i * 2048, 2048)])
            return 0
        lax.fori_loop(0, nz_full, ostep, 0)
        if nz_tail:
            pltpu.sync_copy(accum.at[pl.ds(zbase + nz_full * 2048, nz_tail)],
                            zb.at[pl.ds(0, nz_tail)])
            pltpu.sync_copy(zb.at[pl.ds(0, nz_tail)],
                            out_hbm.at[pl.ds(ob + nz_full * 2048, nz_tail)])

    return kfn


@functools.lru_cache(maxsize=None)
def _make_permpool(n_in):
    nfull = 17
    tail = E2 - nfull * 128

    @functools.partial(
        pl.kernel, mesh=_MESH,
        compiler_params=pltpu.CompilerParams(use_tc_tiling_on_sc=False),
        out_type=[
            jax.ShapeDtypeStruct((2 * 16 * NT, 16), jnp.float32),
            jax.ShapeDtypeStruct((512,), jnp.float32),
        ],
        scratch_types=[
            pltpu.VMEM((128,), jnp.int32),
            pltpu.VMEM((128,), jnp.int32),
            pltpu.VMEM((128, 16), jnp.float32),
            pltpu.VMEM((128, 16), jnp.float32),
            pltpu.VMEM((128, 16), jnp.float32),
            pltpu.VMEM((16,), jnp.float32),
            pltpu.SemaphoreType.DMA,
        ],
    )
    def kfn(x_hbm, srca_hbm, srcb_hbm, out_hbm, g1_hbm,
            sa, sb, ra, rb, vals, g1b, sem):
        c = lax.axis_index("c")
        s = lax.axis_index("s")
        c_off = c * n_in

        def do_chunk(i, nacc, gacc):
            pltpu.sync_copy(srca_hbm.at[pl.ds(s * NT + i * 128, 128)], sa)
            pltpu.sync_copy(srcb_hbm.at[pl.ds(s * NT + i * 128, 128)], sb)
            for g in range(8):
                sl = pl.ds(g * 16, 16)
                sa[sl] = sa[sl] + c_off
                sb[sl] = sb[sl] + c_off
            pltpu.async_copy(x_hbm.at[sa], ra, sem).wait()
            pltpu.async_copy(x_hbm.at[sb], rb, sem).wait()
            for e in range(128):
                row = (ra[e, :] + rb[e, :]) * 0.5
                vals[e, :] = row
                if e < nacc:
                    gacc = gacc + row
            pltpu.sync_copy(
                vals, out_hbm.at[pl.ds((c * 16 + s) * NT + i * 128, 128), :])
            return gacc

        def step(i, gacc):
            return do_chunk(i, 128, gacc)
        gacc = lax.fori_loop(0, nfull, step, jnp.full((16,), 0.0, jnp.float32))
        gacc = do_chunk(nfull, tail, gacc)
        g1b[...] = gacc * (1.0 / E2)
        pltpu.sync_copy(g1b, g1_hbm.at[pl.ds((c * 16 + s) * 16, 16)])

    return kfn


def _comb0_body(a, bb, inv, tc_ref, tp_ref, p_ref, o_ref):
    p = p_ref[...]
    p0 = p[:1123]
    p1 = p[1123:]
    o_ref[...] = (a * tc_ref[...] - p0 - p1 - bb * tp_ref[...]) * inv


def _comb0(tc, tp, p, a, bb, inv):
    body = functools.partial(_comb0_body, a, bb, inv)
    out = pl.pallas_call(
        body, out_shape=jax.ShapeDtypeStruct((1123, 128), jnp.float32),
    )(tc.reshape(1123, 128), tp.reshape(1123, 128), p.reshape(2246, 128))
    return out.reshape(NP0)


def _comb_body(a, bb, inv, tc_ref, tp_ref, p_ref, o_ref):
    o_ref[...] = (a * tc_ref[...] - p_ref[...] - bb * tp_ref[...]) * inv


def _comb16(tc, tp, p, a, bb, inv, rows):
    body = functools.partial(_comb_body, a, bb, inv)
    out = pl.pallas_call(
        body, out_shape=jax.ShapeDtypeStruct((rows, 128), jnp.float32),
    )(tc.reshape(rows, 128), tp.reshape(rows, 128), p.reshape(rows, 128))
    return out.reshape(rows * 8, 16)


def _conv0_body(t0_ref, t1_ref, t2_ref, t3_ref, eev_ref, eod_ref, b_ref,
                o_ref, g_ref):
    i = pl.program_id(1)
    t4 = jnp.concatenate(
        [t0_ref[...], t1_ref[...], t2_ref[...], t3_ref[...]], axis=1)
    bias = b_ref[0][0:1, :]
    ve = jnp.dot(t4, eev_ref[0], preferred_element_type=jnp.float32, precision=jax.lax.Precision.HIGHEST) + bias
    vo = jnp.dot(t4, eod_ref[0], preferred_element_type=jnp.float32, precision=jax.lax.Precision.HIGHEST) + bias
    o = 0.5 * (_leaky(BNS * ve) + _leaky(BNS * vo))
    o_ref[0] = o
    rows = o.shape[0]
    r_iota = lax.broadcasted_iota(jnp.int32, (rows, 128), 0)
    l_iota = lax.broadcasted_iota(jnp.int32, (rows, 128), 1)
    p = (i * rows + r_iota) * 8 + l_iota // 16
    bid = p // E1
    acc = jnp.stack(
        [jnp.where(bid == b, o, 0.0).sum(axis=0) for b in range(16)], axis=0)

    @pl.when(i == 0)
    def _():
        g_ref[0] = acc

    @pl.when(i > 0)
    def _():
        g_ref[0] = g_ref[0] + acc


def _conv0_matmul(t0, t1, t2, t3, eev, eod, b128):
    bm = 1024
    nb = (8978 + bm - 1) // bm
    grid = (2, nb)
    tspec = pl.BlockSpec((bm, 16), lambda h, i: (i, 0))
    espec = pl.BlockSpec((1, 64, 128), lambda h, i: (h, 0, 0))
    return pl.pallas_call(
        _conv0_body,
        grid=grid,
        in_specs=[tspec, tspec, tspec, tspec, espec, espec,
                  pl.BlockSpec((1, 8, 128), lambda h, i: (h, 0, 0))],
        out_specs=[pl.BlockSpec((1, bm, 128), lambda h, i: (h, i, 0)),
                   pl.BlockSpec((1, 16, 128), lambda h, i: (h, 0, 0))],
        out_shape=[jax.ShapeDtypeStruct((2, 8978, 128), jnp.float32),
                   jax.ShapeDtypeStruct((2, 16, 128), jnp.float32)],
    )(t0.reshape(8984, 16), t1.reshape(8984, 16), t2.reshape(8984, 16),
      t3.reshape(8984, 16), eev, eod, b128)


def _conv1_body(th_refs, w_ref, b_ref, o_ref):
    t128 = jnp.concatenate([r[0] for r in th_refs], axis=1)
    z = jnp.dot(t128, w_ref[0], preferred_element_type=jnp.float32, precision=jax.lax.Precision.HIGHEST) + b_ref[0][0:1, :]
    o_ref[0] = _leaky(BNS * z)


def _conv1_matmul(ts, w1rh, b1h):
    bm = 1024
    nb = (N1 + bm - 1) // bm
    grid = (2, nb)
    specs = []
    ops = []
    for t in ts:
        tv = t.reshape(2, N1, 16)
        for h in range(2):
            ops.append(tv)
            specs.append(pl.BlockSpec(
                (1, bm, 16), functools.partial(
                    lambda hh, h_, i_: (hh, i_, 0), h)))
    body = lambda *refs: _conv1_body(refs[:8], refs[8], refs[9], refs[10])
    out = pl.pallas_call(
        body,
        grid=grid,
        in_specs=specs + [pl.BlockSpec((1, 128, 16), lambda h, i: (h, 0, 0)),
                          pl.BlockSpec((1, 8, 16), lambda h, i: (h, 0, 0))],
        out_specs=pl.BlockSpec((1, bm, 16), lambda h, i: (h, i, 0)),
        out_shape=jax.ShapeDtypeStruct((2, N1, 16), jnp.float32),
    )(*ops, w1rh, b1h)
    return out.reshape(2 * N1, 16)


def _conv2_body(refs):
    t128 = jnp.concatenate([r[0] for r in refs[:8]], axis=1)
    z = jnp.dot(t128, refs[8][...], preferred_element_type=jnp.float32, precision=jax.lax.Precision.HIGHEST)
    refs[10][...] = _leaky(BNS * (z + refs[9][...]))


def _conv2_matmul(ts, w128, b2):
    bm = 1024
    nb = (N2 + bm - 1) // bm
    specs = []
    ops = []
    for t in ts:
        tv = t.reshape(2, N2, 16)
        for h in range(2):
            ops.append(tv)
            specs.append(pl.BlockSpec(
                (1, bm, 16), functools.partial(
                    lambda hh, i_: (hh, i_, 0), h)))
    body = lambda *refs: _conv2_body(refs)
    return pl.pallas_call(
        body,
        grid=(nb,),
        in_specs=specs + [pl.BlockSpec((128, 1), lambda i: (0, 0)),
                          pl.BlockSpec((1, 1), lambda i: (0, 0))],
        out_specs=pl.BlockSpec((bm, 1), lambda i: (i, 0)),
        out_shape=jax.ShapeDtypeStruct((N2, 1), jnp.float32),
    )(*ops, w128, b2.reshape(1, 1))


def _mlp_body(x_ref, w1_ref, b1_ref, w2_ref, b2_ref, w3_ref, b3_ref, o_ref):
    x = x_ref[...]
    z = jnp.dot(x, w1_ref[...], preferred_element_type=jnp.float32, precision=jax.lax.Precision.HIGHEST) + b1_ref[...]
    h = jnp.maximum(z * BNS, 0.0)
    z = jnp.dot(h, w2_ref[...], preferred_element_type=jnp.float32, precision=jax.lax.Precision.HIGHEST) + b2_ref[...]
    h = jnp.maximum(z * BNS, 0.0)
    o_ref[...] = jnp.dot(h, w3_ref[...], preferred_element_type=jnp.float32, precision=jax.lax.Precision.HIGHEST) + b3_ref[...]


def _mlp(x, w1, b1, w2, b2, w3, b3):
    return pl.pallas_call(
        _mlp_body,
        out_shape=jax.ShapeDtypeStruct((x.shape[0], 1), jnp.float32),
    )(x, w1, b1[None, :], w2, b2[None, :], w3, b3[None, :])


_MASK_EV = np.zeros((16, 8), np.float32)
_MASK_OD = np.zeros((16, 8), np.float32)
for _j in range(8):
    _MASK_EV[2 * _j, _j] = 1.0
    _MASK_OD[2 * _j + 1, _j] = 1.0


def _conv0_consts(W0, b0):
    w0h = W0[:, 0, :].reshape(4, 2, 16).transpose(1, 0, 2)
    eev = (_MASK_EV[None, None, :, :, None]
           * w0h[:, :, None, None, :]).reshape(2, 64, 128)
    eod = (_MASK_OD[None, None, :, :, None]
           * w0h[:, :, None, None, :]).reshape(2, 64, 128)
    b128 = jnp.tile(b0.reshape(2, 1, 16), (1, 8, 1)).reshape(2, 1, 128)
    b128 = jnp.tile(b128, (1, 8, 1))
    return eev, eod, b128


def kernel(x_s, x_t, edge_index_s, edge_weight_s, edge_index_s1, edge_weight_s1,
           edge_index_s2, edge_weight_s2, idx_dic1,
           W0, b0, W1, b1, W2, b2,
           lin1_W, lin1_b, lin2_W, lin2_b, lin3_W, lin3_b):
    x0 = jnp.pad(x_s[:, 0], (0, NP0 - N0))
    src0, dst0, w0, mp0 = _pad_edges(edge_index_s, edge_weight_s, 32768)
    prop0 = _make_prop_c1(NP0, mp0)

    pa = prop0(x0, src0, dst0, w0)
    t1 = _comb0(x0, x0, pa, 1.0, 0.0, 1.0)
    pb = prop0(t1, src0, dst0, w0)
    t2 = _comb0(t1, x0, pb, 3.0, 1.0, 0.5)
    pc = prop0(t2, src0, dst0, w0)
    t3 = _comb0(t2, t1, pc, 5.0, 2.0, 1.0 / 3.0)

    eev, eod, b128 = _conv0_consts(W0, b0)
    h0p, g0p = _conv0_matmul(x0, t1, t2, t3, eev, eod, b128)
    g0 = (g0p.reshape(2, 16, 8, 16).sum(axis=2) / E1)
    g0 = jnp.concatenate([g0[0], g0[1]], axis=-1)

    x1 = h0p.reshape(2 * N1, 16)
    src1, dst1, w1e, mp1 = _pad_edges(edge_index_s1, edge_weight_s1, 16384)
    prop1 = _make_prop_c16(N1, mp1)
    rows1 = (2 * N1 * 16) // 128

    pa = prop1(x1, src1, dst1, w1e)
    t1 = _comb16(x1.reshape(rows1, 128), x1.reshape(rows1, 128), pa,
                 1.0, 0.0, 1.0, rows1)
    pb = prop1(t1, src1, dst1, w1e)
    t2 = _comb16(t1.reshape(rows1, 128), x1.reshape(rows1, 128), pb,
                 3.0, 1.0, 0.5, rows1)
    pc = prop1(t2, src1, dst1, w1e)
    t3 = _comb16(t2.reshape(rows1, 128), t1.reshape(rows1, 128), pc,
                 5.0, 2.0, 1.0 / 3.0, rows1)

    w1rh = jnp.swapaxes(W1.reshape(128, 2, 16), 0, 1)
    b1h = jnp.tile(b1.reshape(2, 1, 16), (1, 8, 1))
    h1 = _conv1_matmul([x1, t1, t2, t3], w1rh, b1h)

    ev = idx_dic1[0:2 * E2:2]
    od = idx_dic1[1:2 * E2:2]
    boff = jnp.arange(16, dtype=jnp.int32)[:, None] * E1
    srca = jnp.pad((boff + ev[None, :]).astype(jnp.int32),
                   ((0, 0), (0, NT - E2)))
    srcb = jnp.pad((boff + od[None, :]).astype(jnp.int32),
                   ((0, 0), (0, NT - E2)))
    h1p_pad, g1h = _make_permpool(N1)(h1, srca.reshape(-1), srcb.reshape(-1))
    g1h = g1h.reshape(2, 16, 16)
    g1 = jnp.concatenate([g1h[0], g1h[1]], axis=-1)
    x2 = h1p_pad.reshape(2, 16, NT, 16)[:, :, :E2, :].reshape(2 * N2, 16)

    src2, dst2, w2e, mp2 = _pad_edges(edge_index_s2, edge_weight_s2, 16384)
    prop2 = _make_prop_c16(N2, mp2)
    rows2 = (2 * N2 * 16) // 128

    pa = prop2(x2, src2, dst2, w2e)
    t1 = _comb16(x2.reshape(rows2, 128), x2.reshape(rows2, 128), pa,
                 1.0, 0.0, 1.0, rows2)
    pb = prop2(t1, src2, dst2, w2e)
    t2 = _comb16(t1.reshape(rows2, 128), x2.reshape(rows2, 128), pb,
                 3.0, 1.0, 0.5, rows2)
    pc = prop2(t2, src2, dst2, w2e)
    t3 = _comb16(t2.reshape(rows2, 128), t1.reshape(rows2, 128), pc,
                 5.0, 2.0, 1.0 / 3.0, rows2)

    w128 = W2.reshape(128, 1)
    h2 = _conv2_matmul([x2, t1, t2, t3], w128, b2)

    x = jnp.concatenate([h2.reshape(B, E2), g0, g1], axis=-1)
    return _mlp(x, lin1_W, lin1_b, lin2_W, lin2_b, lin3_W, lin3_b)


# speedup vs baseline: 19.4922x; 1.0147x over previous
import functools

import jax
import jax.numpy as jnp
import numpy as np
from jax import lax
from jax.experimental import pallas as pl
from jax.experimental.pallas import tpu as pltpu
from jax.experimental.pallas import tpu_sc as plsc

ROI = 268
E1 = 4489
E2 = 2244
SLOPE = 0.33
BNS = float(1.0 / np.sqrt(1.0 + 1e-5))

B = 16
N0 = B * 8978
N1 = B * E1
N2 = B * E2
NP0 = 16 * 8984
NT = 2304


def _leaky(x):
    return jnp.where(x > 0, x, SLOPE * x)


def _pad_edges(ei, ew, mult):
    m = ei.shape[1]
    mp = ((m + mult - 1) // mult) * mult
    pad = mp - m
    src = jnp.pad(ei[0], (0, pad))
    dst = jnp.pad(ei[1], (0, pad))
    w = jnp.pad(ew, (0, pad))
    return src, dst, w, mp


_MESH = plsc.VectorSubcoreMesh(core_axis_name="c", subcore_axis_name="s")


@functools.lru_cache(maxsize=None)
def _make_prop_c16(n, mp):
    et = mp // 16
    nch = et // 1024
    rpt_a = ((n // 16 + 7) // 8) * 8
    rpt_l = n - 15 * rpt_a

    @functools.partial(
        pl.kernel, mesh=_MESH,
        compiler_params=pltpu.CompilerParams(use_tc_tiling_on_sc=False),
        out_type=jax.ShapeDtypeStruct((2 * n, 16), jnp.float32),
        scratch_types=[
            pltpu.VMEM((1024,), jnp.int32),
            pltpu.VMEM((1024,), jnp.int32),
            pltpu.VMEM((1024,), jnp.float32),
            pltpu.VMEM((1024, 16), jnp.float32),
            pltpu.VMEM((128, 16), jnp.float32),
            pltpu.VMEM_SHARED((n, 16), jnp.float32),
            pltpu.SemaphoreType.DMA,
        ],
    )
    def kfn(x_hbm, src_hbm, dst_hbm, w_hbm, out_hbm,
            srcb, dstb, wb, vals, zb, accum, sem):
        c = lax.axis_index("c")
        s = lax.axis_index("s")
        z16 = jnp.full((16,), 0.0, jnp.float32)
        for r in range(128):
            zb[r, :] = z16
        zbase = s * rpt_a

        def emit_zero(rows):
            nf, tl = rows // 128, rows % 128

            def zstep(i, _):
                pltpu.sync_copy(zb, accum.at[pl.ds(zbase + i * 128, 128), :])
                return 0
            lax.fori_loop(0, nf, zstep, 0)
            if tl:
                pltpu.sync_copy(zb.at[pl.ds(0, tl), :],
                                accum.at[pl.ds(zbase + nf * 128, tl), :])

        @pl.when(s < 15)
        def _():
            emit_zero(rpt_a)

        @pl.when(s == 15)
        def _():
            emit_zero(rpt_l)
        plsc.subcore_barrier()

        e_lo = s * et
        c_off = c * n

        def estep(i, _):
            base = e_lo + i * 1024
            pltpu.sync_copy(src_hbm.at[pl.ds(base, 1024)], srcb)
            pltpu.sync_copy(dst_hbm.at[pl.ds(base, 1024)], dstb)
            pltpu.sync_copy(w_hbm.at[pl.ds(base, 1024)], wb)
            for g in range(64):
                sl = pl.ds(g * 16, 16)
                srcb[sl] = srcb[sl] + c_off
            pltpu.async_copy(x_hbm.at[srcb], vals, sem).wait()

            def scale_j(j, _):
                jb = j * 128
                for g in range(8):
                    wv = wb[pl.ds(jb + g * 16, 16)]
                    for i2 in range(16):
                        vals[jb + g * 16 + i2, :] = (
                            vals[jb + g * 16 + i2, :] * wv[i2])
                return 0
            lax.fori_loop(0, 8, scale_j, 0)
            pltpu.sync_copy(vals, accum.at[dstb], add=True)
            return 0
        lax.fori_loop(0, nch, estep, 0)
        plsc.subcore_barrier()

        obase = c_off + zbase

        def emit_out(rows):
            nf, tl = rows // 128, rows % 128

            def ostep(i, _):
                pltpu.sync_copy(accum.at[pl.ds(zbase + i * 128, 128), :], zb)
                pltpu.sync_copy(zb, out_hbm.at[pl.ds(obase + i * 128, 128), :])
                return 0
            lax.fori_loop(0, nf, ostep, 0)
            if tl:
                pltpu.sync_copy(accum.at[pl.ds(zbase + nf * 128, tl), :],
                                zb.at[pl.ds(0, tl), :])
                pltpu.sync_copy(zb.at[pl.ds(0, tl), :],
                                out_hbm.at[pl.ds(obase + nf * 128, tl), :])

        @pl.when(s < 15)
        def _():
            emit_out(rpt_a)

        @pl.when(s == 15)
        def _():
            emit_out(rpt_l)

    return kfn


@functools.lru_cache(maxsize=None)
def _make_prop_c1(np_len, mp):
    ew_per = mp // 32
    nch = ew_per // 1024
    wpt = np_len // 16
    nz_full, nz_tail = wpt // 2048, wpt % 2048

    @functools.partial(
        pl.kernel, mesh=_MESH,
        compiler_params=pltpu.CompilerParams(use_tc_tiling_on_sc=False),
        out_type=jax.ShapeDtypeStruct((2 * np_len,), jnp.float32),
        scratch_types=[
            pltpu.VMEM((1024,), jnp.int32),
            pltpu.VMEM((1024,), jnp.int32),
            pltpu.VMEM((1024,), jnp.float32),
            pltpu.VMEM((1024,), jnp.float32),
            pltpu.VMEM((2048,), jnp.float32),
            pltpu.VMEM_SHARED((np_len,), jnp.float32),
            pltpu.SemaphoreType.DMA,
        ],
    )
    def kfn(x_hbm, src_hbm, dst_hbm, w_hbm, out_hbm,
            srcb, dstb, wb, vals, zb, accum, sem):
        c = lax.axis_index("c")
        s = lax.axis_index("s")
        z16 = jnp.full((16,), 0.0, jnp.float32)
        for r in range(128):
            zb[pl.ds(r * 16, 16)] = z16
        zbase = s * wpt

        def zstep(i, _):
            pltpu.sync_copy(zb, accum.at[pl.ds(zbase + i * 2048, 2048)])
            return 0
        lax.fori_loop(0, nz_full, zstep, 0)
        if nz_tail:
            pltpu.sync_copy(zb.at[pl.ds(0, nz_tail)],
                            accum.at[pl.ds(zbase + nz_full * 2048, nz_tail)])
        plsc.subcore_barrier()

        e_lo = (c * 16 + s) * ew_per

        def estep(i, _):
            base = e_lo + i * 1024
            pltpu.sync_copy(src_hbm.at[pl.ds(base, 1024)], srcb)
            pltpu.sync_copy(dst_hbm.at[pl.ds(base, 1024)], dstb)
            pltpu.sync_copy(w_hbm.at[pl.ds(base, 1024)], wb)
            pltpu.async_copy(x_hbm.at[srcb], vals, sem).wait()
            for g in range(64):
                sl = pl.ds(g * 16, 16)
                vals[sl] = vals[sl] * wb[sl]
            pltpu.sync_copy(vals, accum.at[dstb], add=True)
            return 0
        lax.fori_loop(0, nch, estep, 0)
        plsc.subcore_barrier()

        ob = c * np_len + zbase

        def ostep(i, _):
            pltpu.sync_copy(accum.at[pl.ds(zbase + i * 2048, 2048)], zb)
            pltpu.sync_copy(zb, out_hbm.at[pl.ds(ob + i * 2048, 2048)])
            return 0
        lax.fori_loop(0, nz_full, ostep, 0)
        if nz_tail:
            pltpu.sync_copy(accum.at[pl.ds(zbase + nz_full * 2048, nz_tail)],
                            zb.at[pl.ds(0, nz_tail)])
            pltpu.sync_copy(zb.at[pl.ds(0, nz_tail)],
                            out_hbm.at[pl.ds(ob + nz_full * 2048, nz_tail)])

    return kfn


@functools.lru_cache(maxsize=None)
def _make_permpool(n_in):
    nfull = 17
    tail = E2 - nfull * 128

    @functools.partial(
        pl.kernel, mesh=_MESH,
        compiler_params=pltpu.CompilerParams(use_tc_tiling_on_sc=False),
        out_type=[
            jax.ShapeDtypeStruct((2 * 16 * NT, 16), jnp.float32),
            jax.ShapeDtypeStruct((512,), jnp.float32),
        ],
        scratch_types=[
            pltpu.VMEM((128,), jnp.int32),
            pltpu.VMEM((128,), jnp.int32),
            pltpu.VMEM((128, 16), jnp.float32),
            pltpu.VMEM((128, 16), jnp.float32),
            pltpu.VMEM((128, 16), jnp.float32),
            pltpu.VMEM((16,), jnp.float32),
            pltpu.SemaphoreType.DMA,
        ],
    )
    def kfn(x_hbm, srca_hbm, srcb_hbm, out_hbm, g1_hbm,
            sa, sb, ra, rb, vals, g1b, sem):
        c = lax.axis_index("c")
        s = lax.axis_index("s")
        c_off = c * n_in

        def do_chunk(i, nacc, gacc):
            pltpu.sync_copy(srca_hbm.at[pl.ds(s * NT + i * 128, 128)], sa)
            pltpu.sync_copy(srcb_hbm.at[pl.ds(s * NT + i * 128, 128)], sb)
            for g in range(8):
                sl = pl.ds(g * 16, 16)
                sa[sl] = sa[sl] + c_off
                sb[sl] = sb[sl] + c_off
            pltpu.async_copy(x_hbm.at[sa], ra, sem).wait()
            pltpu.async_copy(x_hbm.at[sb], rb, sem).wait()
            for e in range(128):
                row = (ra[e, :] + rb[e, :]) * 0.5
                vals[e, :] = row
                if e < nacc:
                    gacc = gacc + row
            pltpu.sync_copy(
                vals, out_hbm.at[pl.ds((c * 16 + s) * NT + i * 128, 128), :])
            return gacc

        def step(i, gacc):
            return do_chunk(i, 128, gacc)
        gacc = lax.fori_loop(0, nfull, step, jnp.full((16,), 0.0, jnp.float32))
        gacc = do_chunk(nfull, tail, gacc)
        g1b[...] = gacc * (1.0 / E2)
        pltpu.sync_copy(g1b, g1_hbm.at[pl.ds((c * 16 + s) * 16, 16)])

    return kfn


def _comb0_body(a, bb, inv, tc_ref, tp_ref, p_ref, o_ref):
    p = p_ref[...]
    p0 = p[:1123]
    p1 = p[1123:]
    o_ref[...] = (a * tc_ref[...] - p0 - p1 - bb * tp_ref[...]) * inv


def _comb0(tc, tp, p, a, bb, inv):
    body = functools.partial(_comb0_body, a, bb, inv)
    out = pl.pallas_call(
        body, out_shape=jax.ShapeDtypeStruct((1123, 128), jnp.float32),
    )(tc.reshape(1123, 128), tp.reshape(1123, 128), p.reshape(2246, 128))
    return out.reshape(NP0)


def _comb_body(a, bb, inv, tc_ref, tp_ref, p_ref, o_ref):
    o_ref[...] = (a * tc_ref[...] - p_ref[...] - bb * tp_ref[...]) * inv


def _comb16(tc, tp, p, a, bb, inv, rows):
    body = functools.partial(_comb_body, a, bb, inv)
    out = pl.pallas_call(
        body, out_shape=jax.ShapeDtypeStruct((rows, 128), jnp.float32),
    )(tc.reshape(rows, 128), tp.reshape(rows, 128), p.reshape(rows, 128))
    return out.reshape(rows * 8, 16)


def _conv0_body(t0_ref, t1_ref, t2_ref, t3_ref, eev_ref, eod_ref, b_ref,
                o_ref, g_ref):
    i = pl.program_id(1)
    t4 = jnp.concatenate(
        [t0_ref[...], t1_ref[...], t2_ref[...], t3_ref[...]], axis=1)
    bias = b_ref[0][0:1, :]
    ve = jnp.dot(t4, eev_ref[0], preferred_element_type=jnp.float32, precision=jax.lax.Precision.HIGHEST) + bias
    vo = jnp.dot(t4, eod_ref[0], preferred_element_type=jnp.float32, precision=jax.lax.Precision.HIGHEST) + bias
    o = 0.5 * (_leaky(BNS * ve) + _leaky(BNS * vo))
    o_ref[0] = o
    rows = o.shape[0]
    r_iota = lax.broadcasted_iota(jnp.int32, (rows, 128), 0)
    l_iota = lax.broadcasted_iota(jnp.int32, (rows, 128), 1)
    p = (i * rows + r_iota) * 8 + l_iota // 16
    bid = p // E1
    acc = jnp.stack(
        [jnp.where(bid == b, o, 0.0).sum(axis=0) for b in range(16)], axis=0)

    @pl.when(i == 0)
    def _():
        g_ref[0] = acc

    @pl.when(i > 0)
    def _():
        g_ref[0] = g_ref[0] + acc


def _conv0_matmul(t0, t1, t2, t3, eev, eod, b128):
    bm = 1024
    nb = (8978 + bm - 1) // bm
    grid = (2, nb)
    tspec = pl.BlockSpec((bm, 16), lambda h, i: (i, 0))
    espec = pl.BlockSpec((1, 64, 128), lambda h, i: (h, 0, 0))
    return pl.pallas_call(
        _conv0_body,
        grid=grid,
        in_specs=[tspec, tspec, tspec, tspec, espec, espec,
                  pl.BlockSpec((1, 8, 128), lambda h, i: (h, 0, 0))],
        out_specs=[pl.BlockSpec((1, bm, 128), lambda h, i: (h, i, 0)),
                   pl.BlockSpec((1, 16, 128), lambda h, i: (h, 0, 0))],
        out_shape=[jax.ShapeDtypeStruct((2, 8978, 128), jnp.float32),
                   jax.ShapeDtypeStruct((2, 16, 128), jnp.float32)],
    )(t0.reshape(8984, 16), t1.reshape(8984, 16), t2.reshape(8984, 16),
      t3.reshape(8984, 16), eev, eod, b128)


def _conv1_body(refs):
    t_refs, wk_ref, b_ref, o_ref = refs[:8], refs[8], refs[9], refs[10]
    full = [jnp.concatenate([t_refs[2 * k][0], t_refs[2 * k + 1][0]], axis=1)
            for k in range(4)]
    z = jnp.dot(full[0], wk_ref[0], preferred_element_type=jnp.float32)
    for k in range(1, 4):
        z = z + jnp.dot(full[k], wk_ref[k], preferred_element_type=jnp.float32)
    z = z + b_ref[0][0:1, :]
    h = _leaky(BNS * z)
    o_ref[0] = h[:, :16]
    o_ref[1] = h[:, 16:]


def _conv1_matmul(ts, W1, b1h):
    bm = 1024
    nb = (N1 + bm - 1) // bm
    specs = []
    ops = []
    for t in ts:
        tv = t.reshape(2, N1, 16)
        for h in range(2):
            ops.append(tv)
            specs.append(pl.BlockSpec(
                (1, bm, 16), functools.partial(
                    lambda hh, i_: (hh, i_, 0), h)))
    out = pl.pallas_call(
        lambda *refs: _conv1_body(refs),
        grid=(nb,),
        in_specs=specs + [pl.BlockSpec((4, 32, 32), lambda i: (0, 0, 0)),
                          pl.BlockSpec((1, 8, 32), lambda i: (0, 0, 0))],
        out_specs=pl.BlockSpec((2, bm, 16), lambda i: (0, i, 0)),
        out_shape=jax.ShapeDtypeStruct((2, N1, 16), jnp.float32),
    )(*ops, W1, b1h)
    return out.reshape(2 * N1, 16)


def _conv2_body(refs):
    t_refs, wk_ref, b_ref, o_ref = refs[:8], refs[8], refs[9], refs[10]
    full = [jnp.concatenate([t_refs[2 * k][0], t_refs[2 * k + 1][0]], axis=1)
            for k in range(4)]
    z = jnp.dot(full[0], wk_ref[0], preferred_element_type=jnp.float32)
    for k in range(1, 4):
        z = z + jnp.dot(full[k], wk_ref[k], preferred_element_type=jnp.float32)
    o_ref[...] = _leaky(BNS * (z + b_ref[...]))


def _conv2_matmul(ts, w128, b2):
    bm = 1024
    nb = (N2 + bm - 1) // bm
    specs = []
    ops = []
    for t in ts:
        tv = t.reshape(2, N2, 16)
        for h in range(2):
            ops.append(tv)
            specs.append(pl.BlockSpec(
                (1, bm, 16), functools.partial(
                    lambda hh, i_: (hh, i_, 0), h)))
    body = lambda *refs: _conv2_body(refs)
    return pl.pallas_call(
        body,
        grid=(nb,),
        in_specs=specs + [pl.BlockSpec((4, 32, 1), lambda i: (0, 0, 0)),
                          pl.BlockSpec((1, 1), lambda i: (0, 0))],
        out_specs=pl.BlockSpec((bm, 1), lambda i: (i, 0)),
        out_shape=jax.ShapeDtypeStruct((N2, 1), jnp.float32),
    )(*ops, w128, b2.reshape(1, 1))


def _mlp_body(x_ref, w1_ref, b1_ref, w2_ref, b2_ref, w3_ref, b3_ref, o_ref):
    x = x_ref[...]
    z = jnp.dot(x, w1_ref[...], preferred_element_type=jnp.float32) + b1_ref[...]
    h = jnp.maximum(z * BNS, 0.0)
    z = jnp.dot(h, w2_ref[...], preferred_element_type=jnp.float32) + b2_ref[...]
    h = jnp.maximum(z * BNS, 0.0)
    o_ref[...] = jnp.dot(h, w3_ref[...], preferred_element_type=jnp.float32) + b3_ref[...]


def _mlp(x, w1, b1, w2, b2, w3, b3):
    return pl.pallas_call(
        _mlp_body,
        out_shape=jax.ShapeDtypeStruct((x.shape[0], 1), jnp.float32),
    )(x, w1, b1[None, :], w2, b2[None, :], w3, b3[None, :])


_MASK_EV = np.zeros((16, 8), np.float32)
_MASK_OD = np.zeros((16, 8), np.float32)
for _j in range(8):
    _MASK_EV[2 * _j, _j] = 1.0
    _MASK_OD[2 * _j + 1, _j] = 1.0


def _conv0_consts(W0, b0):
    w0h = W0[:, 0, :].reshape(4, 2, 16).transpose(1, 0, 2)
    eev = (_MASK_EV[None, None, :, :, None]
           * w0h[:, :, None, None, :]).reshape(2, 64, 128)
    eod = (_MASK_OD[None, None, :, :, None]
           * w0h[:, :, None, None, :]).reshape(2, 64, 128)
    b128 = jnp.tile(b0.reshape(2, 1, 16), (1, 8, 1)).reshape(2, 1, 128)
    b128 = jnp.tile(b128, (1, 8, 1))
    return eev, eod, b128


def kernel(x_s, x_t, edge_index_s, edge_weight_s, edge_index_s1, edge_weight_s1,
           edge_index_s2, edge_weight_s2, idx_dic1,
           W0, b0, W1, b1, W2, b2,
           lin1_W, lin1_b, lin2_W, lin2_b, lin3_W, lin3_b):
    x0 = jnp.pad(x_s[:, 0], (0, NP0 - N0))
    src0, dst0, w0, mp0 = _pad_edges(edge_index_s, edge_weight_s, 32768)
    prop0 = _make_prop_c1(NP0, mp0)

    pa = prop0(x0, src0, dst0, w0)
    t1 = _comb0(x0, x0, pa, 1.0, 0.0, 1.0)
    pb = prop0(t1, src0, dst0, w0)
    t2 = _comb0(t1, x0, pb, 3.0, 1.0, 0.5)
    pc = prop0(t2, src0, dst0, w0)
    t3 = _comb0(t2, t1, pc, 5.0, 2.0, 1.0 / 3.0)

    eev, eod, b128 = _conv0_consts(W0, b0)
    h0p, g0p = _conv0_matmul(x0, t1, t2, t3, eev, eod, b128)
    g0 = (g0p.reshape(2, 16, 8, 16).sum(axis=2) / E1)
    g0 = jnp.concatenate([g0[0], g0[1]], axis=-1)

    x1 = h0p.reshape(2 * N1, 16)
    src1, dst1, w1e, mp1 = _pad_edges(edge_index_s1, edge_weight_s1, 16384)
    prop1 = _make_prop_c16(N1, mp1)
    rows1 = (2 * N1 * 16) // 128

    pa = prop1(x1, src1, dst1, w1e)
    t1 = _comb16(x1.reshape(rows1, 128), x1.reshape(rows1, 128), pa,
                 1.0, 0.0, 1.0, rows1)
    pb = prop1(t1, src1, dst1, w1e)
    t2 = _comb16(t1.reshape(rows1, 128), x1.reshape(rows1, 128), pb,
                 3.0, 1.0, 0.5, rows1)
    pc = prop1(t2, src1, dst1, w1e)
    t3 = _comb16(t2.reshape(rows1, 128), t1.reshape(rows1, 128), pc,
                 5.0, 2.0, 1.0 / 3.0, rows1)

    b1h = jnp.tile(b1.reshape(1, 1, 32), (1, 8, 1))
    h1 = _conv1_matmul([x1, t1, t2, t3], W1, b1h)

    ev = idx_dic1[0:2 * E2:2]
    od = idx_dic1[1:2 * E2:2]
    boff = jnp.arange(16, dtype=jnp.int32)[:, None] * E1
    srca = jnp.pad((boff + ev[None, :]).astype(jnp.int32),
                   ((0, 0), (0, NT - E2)))
    srcb = jnp.pad((boff + od[None, :]).astype(jnp.int32),
                   ((0, 0), (0, NT - E2)))
    h1p_pad, g1h = _make_permpool(N1)(h1, srca.reshape(-1), srcb.reshape(-1))
    g1h = g1h.reshape(2, 16, 16)
    g1 = jnp.concatenate([g1h[0], g1h[1]], axis=-1)
    x2 = h1p_pad.reshape(2, 16, NT, 16)[:, :, :E2, :].reshape(2 * N2, 16)

    src2, dst2, w2e, mp2 = _pad_edges(edge_index_s2, edge_weight_s2, 16384)
    prop2 = _make_prop_c16(N2, mp2)
    rows2 = (2 * N2 * 16) // 128

    pa = prop2(x2, src2, dst2, w2e)
    t1 = _comb16(x2.reshape(rows2, 128), x2.reshape(rows2, 128), pa,
                 1.0, 0.0, 1.0, rows2)
    pb = prop2(t1, src2, dst2, w2e)
    t2 = _comb16(t1.reshape(rows2, 128), x2.reshape(rows2, 128), pb,
                 3.0, 1.0, 0.5, rows2)
    pc = prop2(t2, src2, dst2, w2e)
    t3 = _comb16(t2.reshape(rows2, 128), t1.reshape(rows2, 128), pc,
                 5.0, 2.0, 1.0 / 3.0, rows2)

    h2 = _conv2_matmul([x2, t1, t2, t3], W2, b2)

    x = jnp.concatenate([h2.reshape(B, E2), g0, g1], axis=-1)
    return _mlp(x, lin1_W, lin1_b, lin2_W, lin2_b, lin3_W, lin3_b)
